# baseline (speedup 1.0000x reference)
import jax
import jax.numpy as jnp
from jax.experimental import pallas as pl

_N = 16
_IDX = (0, 136, 272, 409, 545, 682, 818, 955, 1091, 1228, 1364, 1501,
        1637, 1774, 1910, 2047)


def _dot(a, b):
    return jax.lax.dot_general(
        a, b, (((a.ndim - 1,), (0,)), ((), ())),
        preferred_element_type=jnp.float32)


def _lnorm(x, g, b):
    mu = jnp.mean(x, axis=-1, keepdims=True)
    xc = x - mu
    v = jnp.mean(xc * xc, axis=-1, keepdims=True)
    return xc * jax.lax.rsqrt(v + 1e-5) * g + b


def _fused(*refs):
    node_refs = refs[:_N]
    (ne1_w, ne1_b, ne2_w, ne2_b, ne_g, ne_bb,
     ep1_w, ep1_b, ep2_w, ep2_b, ep3_w, ep3_b,
     se1_w, se1_b, se2_w, se2_b,
     g1_w, g1_b, g2_w, g2_b, g3_w, g3_b,
     gr1_w, gr1_b, gr2_w, gr2_b, gr_g, gr_bb,
     gg1_w, gg1_b, gg2_w, gg2_b, gg_g, gg_bb,
     pn1_w, pn1_b, pn2_w, pn2_b,
     sh1a, sh1b, sh1_b, sh2_w, sh2_b, sh_g, sh_bb) = (
        r[...] for r in refs[_N:_N + 45])
    strat_ref, goals_ref, pri_ref, adj_ref, str_ref = refs[_N + 45:]

    nodes = jnp.concatenate([node_refs[k][0, 0] for k in range(_N)], axis=0)

    h = jax.nn.relu(_dot(nodes, ne1_w) + ne1_b)
    h = _dot(h, ne2_w) + ne2_b
    nf = _lnorm(h, ne_g, ne_bb)

    dh = nf.shape[-1]
    pl_ = jnp.broadcast_to(nf[:, None, :], (_N, _N, dh)).reshape(_N * _N, dh)
    pr_ = jnp.broadcast_to(nf[None, :, :], (_N, _N, dh)).reshape(_N * _N, dh)
    pair = jnp.concatenate([pl_, pr_], axis=1)
    e1 = jax.nn.relu(_dot(pair, ep1_w) + ep1_b)
    e2 = jax.nn.relu(_dot(e1, ep2_w) + ep2_b)
    adj_col = jax.nn.sigmoid(_dot(e2, ep3_w) + ep3_b)

    s1 = jax.nn.relu(_dot(pair, se1_w) + se1_b)
    str_col = jnp.tanh(_dot(s1, se2_w) + se2_b)

    r = jax.lax.broadcasted_iota(jnp.int32, (_N * _N, 1), 0)
    is_diag = (r % (_N + 1)) == 0
    offdiag = jnp.where(is_diag, 0.0, 1.0)
    adj_col = adj_col * offdiag
    str_col = str_col * offdiag

    a_col = jnp.where(adj_col > 0.5, 1.0, 0.0) + jnp.where(is_diag, 1.0, 0.0)
    ahat = a_col.reshape(_N, _N)
    deg = jnp.sum(ahat, axis=1, keepdims=True)
    dn = jax.lax.rsqrt(deg)
    x = nf
    for li, (w, b) in enumerate(((g1_w, g1_b), (g2_w, g2_b), (g3_w, g3_b))):
        agg = dn * _dot(ahat, dn * x)
        x = _dot(agg, w) + b
        if li < 2:
            x = jax.nn.relu(x)

    graph = jnp.mean(x, axis=0, keepdims=True)
    g = jax.nn.relu(_dot(graph, gr1_w) + gr1_b)
    g = _dot(g, gr2_w) + gr2_b
    causal = _lnorm(g, gr_g, gr_bb)

    gg = jax.nn.relu(_dot(causal, gg1_w) + gg1_b)
    gg = _dot(gg, gg2_w) + gg2_b
    goals = _lnorm(gg, gg_g, gg_bb)

    pr = jax.nn.relu(_dot(causal, pn1_w) + pn1_b)
    pri = jax.nn.softplus(_dot(pr, pn2_w) + pn2_b)

    sh = jax.nn.relu(_dot(causal, sh1a) + _dot(goals, sh1b) + sh1_b)
    sh = _dot(sh, sh2_w) + sh2_b
    strat = _lnorm(sh, sh_g, sh_bb)

    strat_ref[0] = strat
    goals_ref[0] = goals
    pri_ref[0] = pri
    adj_ref[0] = adj_col
    str_ref[0] = str_col


def kernel(state_buffer, params, step_count, async_interval):
    p = params
    B, S, D = state_buffer.shape
    sb4 = state_buffer.reshape(B, S, 1, D)

    def row(v):
        return v.reshape(1, -1)

    param_args = [
        p['ne1_w'], row(p['ne1_b']), p['ne2_w'], row(p['ne2_b']),
        row(p['ne_ln_g']), row(p['ne_ln_b']),
        p['ep1_w'], row(p['ep1_b']),
        p['ep2_w'], row(p['ep2_b']), p['ep3_w'], row(p['ep3_b']),
        p['se1_w'], row(p['se1_b']),
        p['se2_w'], row(p['se2_b']),
        p['g1_w'], row(p['g1_b']), p['g2_w'], row(p['g2_b']),
        p['g3_w'], row(p['g3_b']),
        p['gr1_w'], row(p['gr1_b']), p['gr2_w'], row(p['gr2_b']),
        row(p['gr_ln_g']), row(p['gr_ln_b']),
        p['gg1_w'], row(p['gg1_b']), p['gg2_w'], row(p['gg2_b']),
        row(p['gg_ln_g']), row(p['gg_ln_b']),
        p['pn1_w'], row(p['pn1_b']), p['pn2_w'], row(p['pn2_b']),
        p['sh1_w'][:64], p['sh1_w'][64:], row(p['sh1_b']),
        p['sh2_w'], row(p['sh2_b']), row(p['sh_ln_g']), row(p['sh_ln_b']),
    ]

    node_specs = [
        pl.BlockSpec((1, 1, 1, D), lambda b, i=i: (b, i, 0, 0))
        for i in _IDX
    ]
    param_specs = [
        pl.BlockSpec(a.shape, lambda b: (0, 0)) for a in param_args
    ]
    out_shape = [
        jax.ShapeDtypeStruct((B, 1, 64), jnp.float32),
        jax.ShapeDtypeStruct((B, 1, 32), jnp.float32),
        jax.ShapeDtypeStruct((B, 1, 1), jnp.float32),
        jax.ShapeDtypeStruct((B, _N * _N, 1), jnp.float32),
        jax.ShapeDtypeStruct((B, _N * _N, 1), jnp.float32),
    ]
    out_specs = [
        pl.BlockSpec((1, 1, 64), lambda b: (b, 0, 0)),
        pl.BlockSpec((1, 1, 32), lambda b: (b, 0, 0)),
        pl.BlockSpec((1, 1, 1), lambda b: (b, 0, 0)),
        pl.BlockSpec((1, _N * _N, 1), lambda b: (b, 0, 0)),
        pl.BlockSpec((1, _N * _N, 1), lambda b: (b, 0, 0)),
    ]

    strat, goals, pri, adj, stren = pl.pallas_call(
        _fused,
        grid=(B,),
        in_specs=node_specs + param_specs,
        out_specs=out_specs,
        out_shape=out_shape,
    )(*([sb4] * _N + param_args))

    strategic = strat[:, 0, :]
    goals = goals[:, 0, :]
    priorities = pri[:, 0, :]
    adjacency = adj.reshape(B, _N, _N)
    strengths = stren.reshape(B, _N, _N)

    rem = jnp.asarray(step_count) % jnp.asarray(async_interval)
    active = rem == 0
    nan = jnp.float32(jnp.nan)
    gate = lambda o: jnp.where(active, o, nan)
    return (gate(strategic), gate(goals), gate(priorities),
            gate(adjacency), gate(strengths))


# speedup vs baseline: 1.6729x; 1.6729x over previous
import jax
import jax.numpy as jnp
from jax.experimental import pallas as pl

_N = 16
_B = 32
_IDX = (0, 136, 272, 409, 545, 682, 818, 955, 1091, 1228, 1364, 1501,
        1637, 1774, 1910, 2047)


def _dot(a, b):
    return jax.lax.dot_general(
        a, b, (((a.ndim - 1,), (0,)), ((), ())),
        preferred_element_type=jnp.float32)


def _lnorm(x, g, b):
    mu = jnp.mean(x, axis=-1, keepdims=True)
    xc = x - mu
    v = jnp.mean(xc * xc, axis=-1, keepdims=True)
    return xc * jax.lax.rsqrt(v + 1e-5) * g + b


def _fused(*refs):
    node_refs = refs[:_N]
    (ne1_w, ne1_b, ne2_w, ne2_b, ne_g, ne_bb,
     ep1_w, ep1_b, ep2_w, ep2_b, ep3_w, ep3_b,
     se1_w, se1_b, se2_w, se2_b,
     g1_w, g1_b, g2_w, g2_b, g3_w, g3_b,
     gr1_w, gr1_b, gr2_w, gr2_b, gr_g, gr_bb,
     gg1_w, gg1_b, gg2_w, gg2_b, gg_g, gg_bb,
     pn1_w, pn1_b, pn2_w, pn2_b,
     sh1a, sh1b, sh1_b, sh2_w, sh2_b, sh_g, sh_bb) = (
        r[...] for r in refs[_N:_N + 45])
    strat_ref, goals_ref, pri_ref, adj_ref, str_ref = refs[_N + 45:]

    BN = _B * _N
    cols = [node_refs[k][:, 0, 0, :] for k in range(_N)]
    nodes = jnp.concatenate([c[:, None, :] for c in cols], axis=1)
    nodes = nodes.reshape(BN, nodes.shape[-1])

    h = jax.nn.relu(_dot(nodes, ne1_w) + ne1_b)
    h = _dot(h, ne2_w) + ne2_b
    nf = _lnorm(h, ne_g, ne_bb)
    dh = nf.shape[-1]

    nf3 = nf.reshape(_B, _N, dh)
    left = jnp.broadcast_to(nf3[:, :, None, :], (_B, _N, _N, dh))
    right = jnp.broadcast_to(nf3[:, None, :, :], (_B, _N, _N, dh))
    pair = jnp.concatenate([left.reshape(_B * _N * _N, dh),
                            right.reshape(_B * _N * _N, dh)], axis=1)
    e1 = jax.nn.relu(_dot(pair, ep1_w) + ep1_b)
    e2 = jax.nn.relu(_dot(e1, ep2_w) + ep2_b)
    adj_col = jax.nn.sigmoid(_dot(e2, ep3_w) + ep3_b)
    s1 = jax.nn.relu(_dot(pair, se1_w) + se1_b)
    str_col = jnp.tanh(_dot(s1, se2_w) + se2_b)

    rr = jax.lax.broadcasted_iota(jnp.int32, (_B * _N * _N, 1), 0) % (_N * _N)
    is_diag = (rr % (_N + 1)) == 0
    offdiag = jnp.where(is_diag, 0.0, 1.0)
    adj_col = adj_col * offdiag
    str_col = str_col * offdiag

    a_col = jnp.where(adj_col > 0.5, 1.0, 0.0) + jnp.where(is_diag, 1.0, 0.0)
    m = a_col.reshape(BN, _N)
    deg = jnp.sum(m, axis=1, keepdims=True)
    dn = jax.lax.rsqrt(deg)
    tiled = jnp.broadcast_to(m[:, None, :], (BN, _B, _N)).reshape(BN, BN)
    rowg = jax.lax.broadcasted_iota(jnp.int32, (BN, BN), 0) // _N
    colg = jax.lax.broadcasted_iota(jnp.int32, (BN, BN), 1) // _N
    bd = jnp.where(rowg == colg, tiled, 0.0)

    x = nf
    for li, (w, b) in enumerate(((g1_w, g1_b), (g2_w, g2_b), (g3_w, g3_b))):
        agg = dn * _dot(bd, dn * x)
        x = _dot(agg, w) + b
        if li < 2:
            x = jax.nn.relu(x)

    graph = jnp.mean(x.reshape(_B, _N, x.shape[-1]), axis=1)
    g = jax.nn.relu(_dot(graph, gr1_w) + gr1_b)
    g = _dot(g, gr2_w) + gr2_b
    causal = _lnorm(g, gr_g, gr_bb)

    gg = jax.nn.relu(_dot(causal, gg1_w) + gg1_b)
    gg = _dot(gg, gg2_w) + gg2_b
    goals = _lnorm(gg, gg_g, gg_bb)

    pr = jax.nn.relu(_dot(causal, pn1_w) + pn1_b)
    pri = jax.nn.softplus(_dot(pr, pn2_w) + pn2_b)

    sh = jax.nn.relu(_dot(causal, sh1a) + _dot(goals, sh1b) + sh1_b)
    sh = _dot(sh, sh2_w) + sh2_b
    strat = _lnorm(sh, sh_g, sh_bb)

    strat_ref[...] = strat
    goals_ref[...] = goals
    pri_ref[...] = pri
    adj_ref[...] = adj_col
    str_ref[...] = str_col


def kernel(state_buffer, params, step_count, async_interval):
    p = params
    B, S, D = state_buffer.shape
    sb4 = state_buffer.reshape(B, S, 1, D)

    def row(v):
        return v.reshape(1, -1)

    param_args = [
        p['ne1_w'], row(p['ne1_b']), p['ne2_w'], row(p['ne2_b']),
        row(p['ne_ln_g']), row(p['ne_ln_b']),
        p['ep1_w'], row(p['ep1_b']),
        p['ep2_w'], row(p['ep2_b']), p['ep3_w'], row(p['ep3_b']),
        p['se1_w'], row(p['se1_b']),
        p['se2_w'], row(p['se2_b']),
        p['g1_w'], row(p['g1_b']), p['g2_w'], row(p['g2_b']),
        p['g3_w'], row(p['g3_b']),
        p['gr1_w'], row(p['gr1_b']), p['gr2_w'], row(p['gr2_b']),
        row(p['gr_ln_g']), row(p['gr_ln_b']),
        p['gg1_w'], row(p['gg1_b']), p['gg2_w'], row(p['gg2_b']),
        row(p['gg_ln_g']), row(p['gg_ln_b']),
        p['pn1_w'], row(p['pn1_b']), p['pn2_w'], row(p['pn2_b']),
        p['sh1_w'][:64], p['sh1_w'][64:], row(p['sh1_b']),
        p['sh2_w'], row(p['sh2_b']), row(p['sh_ln_g']), row(p['sh_ln_b']),
    ]

    node_specs = [
        pl.BlockSpec((B, 1, 1, D), lambda i=0, *, _r=r: (0, _r, 0, 0))
        for r in _IDX
    ]
    param_specs = [
        pl.BlockSpec(a.shape, lambda i: (0, 0)) for a in param_args
    ]
    out_shape = [
        jax.ShapeDtypeStruct((B, 64), jnp.float32),
        jax.ShapeDtypeStruct((B, 32), jnp.float32),
        jax.ShapeDtypeStruct((B, 1), jnp.float32),
        jax.ShapeDtypeStruct((B * _N * _N, 1), jnp.float32),
        jax.ShapeDtypeStruct((B * _N * _N, 1), jnp.float32),
    ]
    out_specs = [
        pl.BlockSpec((B, 64), lambda i: (0, 0)),
        pl.BlockSpec((B, 32), lambda i: (0, 0)),
        pl.BlockSpec((B, 1), lambda i: (0, 0)),
        pl.BlockSpec((B * _N * _N, 1), lambda i: (0, 0)),
        pl.BlockSpec((B * _N * _N, 1), lambda i: (0, 0)),
    ]

    strat, goals, pri, adj, stren = pl.pallas_call(
        _fused,
        grid=(1,),
        in_specs=node_specs + param_specs,
        out_specs=out_specs,
        out_shape=out_shape,
    )(*([sb4] * _N + param_args))

    adjacency = adj.reshape(B, _N, _N)
    strengths = stren.reshape(B, _N, _N)

    rem = jnp.asarray(step_count) % jnp.asarray(async_interval)
    active = rem == 0
    nan = jnp.float32(jnp.nan)
    gate = lambda o: jnp.where(active, o, nan)
    return (gate(strat), gate(goals), gate(pri),
            gate(adjacency), gate(strengths))


# speedup vs baseline: 4.9701x; 2.9710x over previous
import jax
import jax.numpy as jnp
from jax.experimental import pallas as pl

_N = 16
_B = 32
_IDX = (0, 136, 272, 409, 545, 682, 818, 955, 1091, 1228, 1364, 1501,
        1637, 1774, 1910, 2047)


def _dot(a, b):
    return jax.lax.dot_general(
        a, b, (((a.ndim - 1,), (0,)), ((), ())),
        preferred_element_type=jnp.float32)


def _lnorm(x, g, b):
    mu = jnp.mean(x, axis=-1, keepdims=True)
    xc = x - mu
    v = jnp.mean(xc * xc, axis=-1, keepdims=True)
    return xc * jax.lax.rsqrt(v + 1e-5) * g + b


def _fused(*refs):
    node_refs = refs[:_N]
    (ne1_w, ne1_b, ne2_w, ne2_b, ne_g, ne_bb,
     ep1_w, ep1_b, ep2_w, ep2_b, ep3_w, ep3_b,
     se1_w, se1_b, se2_w, se2_b,
     g1_w, g1_b, g2_w, g2_b, g3_w, g3_b,
     gr1_w, gr1_b, gr2_w, gr2_b, gr_g, gr_bb,
     gg1_w, gg1_b, gg2_w, gg2_b, gg_g, gg_bb,
     pn1_w, pn1_b, pn2_w, pn2_b,
     sh1a, sh1b, sh1_b, sh2_w, sh2_b, sh_g, sh_bb) = (
        r[...] for r in refs[_N:_N + 45])
    strat_ref, goals_ref, pri_ref, adj_ref, str_ref = refs[_N + 45:]

    BN = _B * _N
    cols = [node_refs[k][:, _IDX[k] % 8, :] for k in range(_N)]
    nodes = jnp.concatenate([c[:, None, :] for c in cols], axis=1)
    nodes = nodes.reshape(BN, nodes.shape[-1])

    h = jax.nn.relu(_dot(nodes, ne1_w) + ne1_b)
    h = _dot(h, ne2_w) + ne2_b
    nf = _lnorm(h, ne_g, ne_bb)
    dh = nf.shape[-1]

    nf3 = nf.reshape(_B, _N, dh)
    left = jnp.broadcast_to(nf3[:, :, None, :], (_B, _N, _N, dh))
    right = jnp.broadcast_to(nf3[:, None, :, :], (_B, _N, _N, dh))
    pair = jnp.concatenate([left.reshape(_B * _N * _N, dh),
                            right.reshape(_B * _N * _N, dh)], axis=1)
    e1 = jax.nn.relu(_dot(pair, ep1_w) + ep1_b)
    e2 = jax.nn.relu(_dot(e1, ep2_w) + ep2_b)
    adj_col = jax.nn.sigmoid(_dot(e2, ep3_w) + ep3_b)
    s1 = jax.nn.relu(_dot(pair, se1_w) + se1_b)
    str_col = jnp.tanh(_dot(s1, se2_w) + se2_b)

    rr = jax.lax.broadcasted_iota(jnp.int32, (_B * _N * _N, 1), 0) % (_N * _N)
    is_diag = (rr % (_N + 1)) == 0
    offdiag = jnp.where(is_diag, 0.0, 1.0)
    adj_col = adj_col * offdiag
    str_col = str_col * offdiag

    a_col = jnp.where(adj_col > 0.5, 1.0, 0.0) + jnp.where(is_diag, 1.0, 0.0)
    m = a_col.reshape(BN, _N)
    deg = jnp.sum(m, axis=1, keepdims=True)
    dn = jax.lax.rsqrt(deg)
    tiled = jnp.broadcast_to(m[:, None, :], (BN, _B, _N)).reshape(BN, BN)
    rowg = jax.lax.broadcasted_iota(jnp.int32, (BN, BN), 0) // _N
    colg = jax.lax.broadcasted_iota(jnp.int32, (BN, BN), 1) // _N
    bd = jnp.where(rowg == colg, tiled, 0.0)

    x = nf
    for li, (w, b) in enumerate(((g1_w, g1_b), (g2_w, g2_b), (g3_w, g3_b))):
        agg = dn * _dot(bd, dn * x)
        x = _dot(agg, w) + b
        if li < 2:
            x = jax.nn.relu(x)

    graph = jnp.mean(x.reshape(_B, _N, x.shape[-1]), axis=1)
    g = jax.nn.relu(_dot(graph, gr1_w) + gr1_b)
    g = _dot(g, gr2_w) + gr2_b
    causal = _lnorm(g, gr_g, gr_bb)

    gg = jax.nn.relu(_dot(causal, gg1_w) + gg1_b)
    gg = _dot(gg, gg2_w) + gg2_b
    goals = _lnorm(gg, gg_g, gg_bb)

    pr = jax.nn.relu(_dot(causal, pn1_w) + pn1_b)
    pri = jax.nn.softplus(_dot(pr, pn2_w) + pn2_b)

    sh = jax.nn.relu(_dot(causal, sh1a) + _dot(goals, sh1b) + sh1_b)
    sh = _dot(sh, sh2_w) + sh2_b
    strat = _lnorm(sh, sh_g, sh_bb)

    strat_ref[...] = strat
    goals_ref[...] = goals
    pri_ref[...] = pri
    adj_ref[...] = adj_col
    str_ref[...] = str_col


def kernel(state_buffer, params, step_count, async_interval):
    p = params
    B, S, D = state_buffer.shape

    def row(v):
        return v.reshape(1, -1)

    param_args = [
        p['ne1_w'], row(p['ne1_b']), p['ne2_w'], row(p['ne2_b']),
        row(p['ne_ln_g']), row(p['ne_ln_b']),
        p['ep1_w'], row(p['ep1_b']),
        p['ep2_w'], row(p['ep2_b']), p['ep3_w'], row(p['ep3_b']),
        p['se1_w'], row(p['se1_b']),
        p['se2_w'], row(p['se2_b']),
        p['g1_w'], row(p['g1_b']), p['g2_w'], row(p['g2_b']),
        p['g3_w'], row(p['g3_b']),
        p['gr1_w'], row(p['gr1_b']), p['gr2_w'], row(p['gr2_b']),
        row(p['gr_ln_g']), row(p['gr_ln_b']),
        p['gg1_w'], row(p['gg1_b']), p['gg2_w'], row(p['gg2_b']),
        row(p['gg_ln_g']), row(p['gg_ln_b']),
        p['pn1_w'], row(p['pn1_b']), p['pn2_w'], row(p['pn2_b']),
        p['sh1_w'][:64], p['sh1_w'][64:], row(p['sh1_b']),
        p['sh2_w'], row(p['sh2_b']), row(p['sh_ln_g']), row(p['sh_ln_b']),
    ]

    node_specs = [
        pl.BlockSpec((B, 8, D), lambda i=0, *, _r=r: (0, _r // 8, 0))
        for r in _IDX
    ]
    param_specs = [
        pl.BlockSpec(a.shape, lambda i: (0, 0)) for a in param_args
    ]
    out_shape = [
        jax.ShapeDtypeStruct((B, 64), jnp.float32),
        jax.ShapeDtypeStruct((B, 32), jnp.float32),
        jax.ShapeDtypeStruct((B, 1), jnp.float32),
        jax.ShapeDtypeStruct((B * _N * _N, 1), jnp.float32),
        jax.ShapeDtypeStruct((B * _N * _N, 1), jnp.float32),
    ]
    out_specs = [
        pl.BlockSpec((B, 64), lambda i: (0, 0)),
        pl.BlockSpec((B, 32), lambda i: (0, 0)),
        pl.BlockSpec((B, 1), lambda i: (0, 0)),
        pl.BlockSpec((B * _N * _N, 1), lambda i: (0, 0)),
        pl.BlockSpec((B * _N * _N, 1), lambda i: (0, 0)),
    ]

    strat, goals, pri, adj, stren = pl.pallas_call(
        _fused,
        grid=(1,),
        in_specs=node_specs + param_specs,
        out_specs=out_specs,
        out_shape=out_shape,
    )(*([state_buffer] * _N + param_args))

    adjacency = adj.reshape(B, _N, _N)
    strengths = stren.reshape(B, _N, _N)

    rem = jnp.asarray(step_count) % jnp.asarray(async_interval)
    active = rem == 0
    nan = jnp.float32(jnp.nan)
    gate = lambda o: jnp.where(active, o, nan)
    return (gate(strat), gate(goals), gate(pri),
            gate(adjacency), gate(strengths))


# speedup vs baseline: 5.8432x; 1.1757x over previous
import jax
import jax.numpy as jnp
from jax.experimental import pallas as pl

_N = 16
_B = 32
_IDX = (0, 136, 272, 409, 545, 682, 818, 955, 1091, 1228, 1364, 1501,
        1637, 1774, 1910, 2047)


def _dot(a, b):
    return jax.lax.dot_general(
        a, b, (((a.ndim - 1,), (0,)), ((), ())),
        preferred_element_type=jnp.float32)


def _lnorm(x, g, b):
    mu = jnp.mean(x, axis=-1, keepdims=True)
    xc = x - mu
    v = jnp.mean(xc * xc, axis=-1, keepdims=True)
    return xc * jax.lax.rsqrt(v + 1e-5) * g + b


def _fused(*refs):
    node_refs = refs[:_N]
    (ne1_w, ne1_b, ne2_w, ne2_b, ne_g, ne_bb,
     ep1_w, ep1_b, ep2_w, ep2_b, ep3_w, ep3_b,
     se1_w, se1_b, se2_w, se2_b,
     g1_w, g1_b, g2_w, g2_b, g3_w, g3_b,
     gr1_w, gr1_b, gr2_w, gr2_b, gr_g, gr_bb,
     gg1_w, gg1_b, gg2_w, gg2_b, gg_g, gg_bb,
     pn1_w, pn1_b, pn2_w, pn2_b,
     sh1a, sh1b, sh1_b, sh2_w, sh2_b, sh_g, sh_bb) = (
        r[...] for r in refs[_N:_N + 45])
    act_ref = refs[_N + 45]
    strat_ref, goals_ref, pri_ref, adj_ref, str_ref = refs[_N + 46:]

    BN = _B * _N
    cols = [node_refs[k][:, _IDX[k] % 8, :] for k in range(_N)]
    nodes = jnp.concatenate([c[:, None, :] for c in cols], axis=1)
    nodes = nodes.reshape(BN, nodes.shape[-1])

    h = jax.nn.relu(_dot(nodes, ne1_w) + ne1_b)
    h = _dot(h, ne2_w) + ne2_b
    nf = _lnorm(h, ne_g, ne_bb)
    dh = nf.shape[-1]

    nf3 = nf.reshape(_B, _N, dh)
    left = jnp.broadcast_to(nf3[:, :, None, :], (_B, _N, _N, dh))
    right = jnp.broadcast_to(nf3[:, None, :, :], (_B, _N, _N, dh))
    pair = jnp.concatenate([left.reshape(_B * _N * _N, dh),
                            right.reshape(_B * _N * _N, dh)], axis=1)
    e1 = jax.nn.relu(_dot(pair, ep1_w) + ep1_b)
    e2 = jax.nn.relu(_dot(e1, ep2_w) + ep2_b)
    adj_col = jax.nn.sigmoid(_dot(e2, ep3_w) + ep3_b)
    s1 = jax.nn.relu(_dot(pair, se1_w) + se1_b)
    str_col = jnp.tanh(_dot(s1, se2_w) + se2_b)

    rr = jax.lax.broadcasted_iota(jnp.int32, (_B * _N * _N, 1), 0) % (_N * _N)
    is_diag = (rr % (_N + 1)) == 0
    offdiag = jnp.where(is_diag, 0.0, 1.0)
    adj_col = adj_col * offdiag
    str_col = str_col * offdiag

    a_col = jnp.where(adj_col > 0.5, 1.0, 0.0) + jnp.where(is_diag, 1.0, 0.0)
    m = a_col.reshape(BN, _N)
    deg = jnp.sum(m, axis=1, keepdims=True)
    dn = jax.lax.rsqrt(deg)
    tiled = jnp.broadcast_to(m[:, None, :], (BN, _B, _N)).reshape(BN, BN)
    rowg = jax.lax.broadcasted_iota(jnp.int32, (BN, BN), 0) // _N
    colg = jax.lax.broadcasted_iota(jnp.int32, (BN, BN), 1) // _N
    bd = jnp.where(rowg == colg, tiled, 0.0)

    x = nf
    for li, (w, b) in enumerate(((g1_w, g1_b), (g2_w, g2_b), (g3_w, g3_b))):
        agg = dn * _dot(bd, dn * x)
        x = _dot(agg, w) + b
        if li < 2:
            x = jax.nn.relu(x)

    graph = jnp.mean(x.reshape(_B, _N, x.shape[-1]), axis=1)
    g = jax.nn.relu(_dot(graph, gr1_w) + gr1_b)
    g = _dot(g, gr2_w) + gr2_b
    causal = _lnorm(g, gr_g, gr_bb)

    gg = jax.nn.relu(_dot(causal, gg1_w) + gg1_b)
    gg = _dot(gg, gg2_w) + gg2_b
    goals = _lnorm(gg, gg_g, gg_bb)

    pr = jax.nn.relu(_dot(causal, pn1_w) + pn1_b)
    pri = jax.nn.softplus(_dot(pr, pn2_w) + pn2_b)

    sh = jax.nn.relu(_dot(causal, sh1a) + _dot(goals, sh1b) + sh1_b)
    sh = _dot(sh, sh2_w) + sh2_b
    strat = _lnorm(sh, sh_g, sh_bb)

    active = act_ref[0, 0] == 1

    def gate(o):
        return jnp.where(active, o, jnp.full_like(o, jnp.nan))

    strat_ref[...] = gate(strat)
    goals_ref[...] = gate(goals)
    pri_ref[...] = gate(pri)
    adj_ref[...] = gate(adj_col.reshape(_B, _N, _N))
    str_ref[...] = gate(str_col.reshape(_B, _N, _N))


def kernel(state_buffer, params, step_count, async_interval):
    p = params
    B, S, D = state_buffer.shape

    def row(v):
        return v.reshape(1, -1)

    param_args = [
        p['ne1_w'], row(p['ne1_b']), p['ne2_w'], row(p['ne2_b']),
        row(p['ne_ln_g']), row(p['ne_ln_b']),
        p['ep1_w'], row(p['ep1_b']),
        p['ep2_w'], row(p['ep2_b']), p['ep3_w'], row(p['ep3_b']),
        p['se1_w'], row(p['se1_b']),
        p['se2_w'], row(p['se2_b']),
        p['g1_w'], row(p['g1_b']), p['g2_w'], row(p['g2_b']),
        p['g3_w'], row(p['g3_b']),
        p['gr1_w'], row(p['gr1_b']), p['gr2_w'], row(p['gr2_b']),
        row(p['gr_ln_g']), row(p['gr_ln_b']),
        p['gg1_w'], row(p['gg1_b']), p['gg2_w'], row(p['gg2_b']),
        row(p['gg_ln_g']), row(p['gg_ln_b']),
        p['pn1_w'], row(p['pn1_b']), p['pn2_w'], row(p['pn2_b']),
        p['sh1_w'][:64], p['sh1_w'][64:], row(p['sh1_b']),
        p['sh2_w'], row(p['sh2_b']), row(p['sh_ln_g']), row(p['sh_ln_b']),
    ]

    node_specs = [
        pl.BlockSpec((B, 8, D), lambda i=0, *, _r=r: (0, _r // 8, 0))
        for r in _IDX
    ]
    rem = jnp.asarray(step_count) % jnp.asarray(async_interval)
    act = (rem == 0).astype(jnp.int32).reshape(1, 1)

    param_specs = [
        pl.BlockSpec(a.shape, lambda i: (0, 0)) for a in param_args
    ] + [pl.BlockSpec((1, 1), lambda i: (0, 0))]
    out_shape = [
        jax.ShapeDtypeStruct((B, 64), jnp.float32),
        jax.ShapeDtypeStruct((B, 32), jnp.float32),
        jax.ShapeDtypeStruct((B, 1), jnp.float32),
        jax.ShapeDtypeStruct((B, _N, _N), jnp.float32),
        jax.ShapeDtypeStruct((B, _N, _N), jnp.float32),
    ]
    out_specs = [
        pl.BlockSpec((B, 64), lambda i: (0, 0)),
        pl.BlockSpec((B, 32), lambda i: (0, 0)),
        pl.BlockSpec((B, 1), lambda i: (0, 0)),
        pl.BlockSpec((B, _N, _N), lambda i: (0, 0, 0)),
        pl.BlockSpec((B, _N, _N), lambda i: (0, 0, 0)),
    ]

    outs = pl.pallas_call(
        _fused,
        grid=(1,),
        in_specs=node_specs + param_specs,
        out_specs=out_specs,
        out_shape=out_shape,
    )(*([state_buffer] * _N + param_args + [act]))
    return tuple(outs)


# speedup vs baseline: 5.9932x; 1.0257x over previous
import jax
import jax.numpy as jnp
from jax.experimental import pallas as pl
from jax.experimental.pallas import tpu as pltpu

_N = 16
_B = 32
_IDX = (0, 136, 272, 409, 545, 682, 818, 955, 1091, 1228, 1364, 1501,
        1637, 1774, 1910, 2047)


def _dot(a, b):
    return jax.lax.dot_general(
        a, b, (((a.ndim - 1,), (0,)), ((), ())),
        preferred_element_type=jnp.float32)


def _lnorm(x, g, b):
    mu = jnp.mean(x, axis=-1, keepdims=True)
    xc = x - mu
    v = jnp.mean(xc * xc, axis=-1, keepdims=True)
    return xc * jax.lax.rsqrt(v + 1e-5) * g + b


def _fused(*refs):
    sb_ref = refs[0]
    (ne1_w, ne1_b, ne2_w, ne2_b, ne_g, ne_bb,
     ep1_w, ep1_b, ep2_w, ep2_b, ep3_w, ep3_b,
     se1_w, se1_b, se2_w, se2_b,
     g1_w, g1_b, g2_w, g2_b, g3_w, g3_b,
     gr1_w, gr1_b, gr2_w, gr2_b, gr_g, gr_bb,
     gg1_w, gg1_b, gg2_w, gg2_b, gg_g, gg_bb,
     pn1_w, pn1_b, pn2_w, pn2_b,
     sh1a, sh1b, sh1_b, sh2_w, sh2_b, sh_g, sh_bb) = (
        r[...] for r in refs[1:1 + 45])
    act_ref = refs[1 + 45]
    strat_ref, goals_ref, pri_ref, adj_ref, str_ref = refs[1 + 46:1 + 51]
    nodes_scr, sem = refs[1 + 51:]

    BN = _B * _N
    copies = [
        pltpu.make_async_copy(
            sb_ref.at[:, pl.ds(_IDX[k], 1), :],
            nodes_scr.at[:, pl.ds(k, 1), :],
            sem,
        )
        for k in range(_N)
    ]
    for c in copies:
        c.start()
    for c in copies:
        c.wait()
    nodes = nodes_scr[...].reshape(BN, sb_ref.shape[-1])

    h = jax.nn.relu(_dot(nodes, ne1_w) + ne1_b)
    h = _dot(h, ne2_w) + ne2_b
    nf = _lnorm(h, ne_g, ne_bb)
    dh = nf.shape[-1]

    nf3 = nf.reshape(_B, _N, dh)
    left = jnp.broadcast_to(nf3[:, :, None, :], (_B, _N, _N, dh))
    right = jnp.broadcast_to(nf3[:, None, :, :], (_B, _N, _N, dh))
    pair = jnp.concatenate([left.reshape(_B * _N * _N, dh),
                            right.reshape(_B * _N * _N, dh)], axis=1)
    e1 = jax.nn.relu(_dot(pair, ep1_w) + ep1_b)
    e2 = jax.nn.relu(_dot(e1, ep2_w) + ep2_b)
    adj_col = jax.nn.sigmoid(_dot(e2, ep3_w) + ep3_b)
    s1 = jax.nn.relu(_dot(pair, se1_w) + se1_b)
    str_col = jnp.tanh(_dot(s1, se2_w) + se2_b)

    rr = jax.lax.broadcasted_iota(jnp.int32, (_B * _N * _N, 1), 0) % (_N * _N)
    is_diag = (rr % (_N + 1)) == 0
    offdiag = jnp.where(is_diag, 0.0, 1.0)
    adj_col = adj_col * offdiag
    str_col = str_col * offdiag

    a_col = jnp.where(adj_col > 0.5, 1.0, 0.0) + jnp.where(is_diag, 1.0, 0.0)
    m = a_col.reshape(BN, _N)
    deg = jnp.sum(m, axis=1, keepdims=True)
    dn = jax.lax.rsqrt(deg)
    tiled = jnp.broadcast_to(m[:, None, :], (BN, _B, _N)).reshape(BN, BN)
    rowg = jax.lax.broadcasted_iota(jnp.int32, (BN, BN), 0) // _N
    colg = jax.lax.broadcasted_iota(jnp.int32, (BN, BN), 1) // _N
    bd = jnp.where(rowg == colg, tiled, 0.0)

    x = nf
    for li, (w, b) in enumerate(((g1_w, g1_b), (g2_w, g2_b), (g3_w, g3_b))):
        agg = dn * _dot(bd, dn * x)
        x = _dot(agg, w) + b
        if li < 2:
            x = jax.nn.relu(x)

    graph = jnp.mean(x.reshape(_B, _N, x.shape[-1]), axis=1)
    g = jax.nn.relu(_dot(graph, gr1_w) + gr1_b)
    g = _dot(g, gr2_w) + gr2_b
    causal = _lnorm(g, gr_g, gr_bb)

    gg = jax.nn.relu(_dot(causal, gg1_w) + gg1_b)
    gg = _dot(gg, gg2_w) + gg2_b
    goals = _lnorm(gg, gg_g, gg_bb)

    pr = jax.nn.relu(_dot(causal, pn1_w) + pn1_b)
    pri = jax.nn.softplus(_dot(pr, pn2_w) + pn2_b)

    sh = jax.nn.relu(_dot(causal, sh1a) + _dot(goals, sh1b) + sh1_b)
    sh = _dot(sh, sh2_w) + sh2_b
    strat = _lnorm(sh, sh_g, sh_bb)

    active = act_ref[0, 0] == 1

    def gate(o):
        return jnp.where(active, o, jnp.full_like(o, jnp.nan))

    strat_ref[...] = gate(strat)
    goals_ref[...] = gate(goals)
    pri_ref[...] = gate(pri)
    adj_ref[...] = gate(adj_col.reshape(_B, _N, _N))
    str_ref[...] = gate(str_col.reshape(_B, _N, _N))


def kernel(state_buffer, params, step_count, async_interval):
    p = params
    B, S, D = state_buffer.shape

    def row(v):
        return v.reshape(1, -1)

    param_args = [
        p['ne1_w'], row(p['ne1_b']), p['ne2_w'], row(p['ne2_b']),
        row(p['ne_ln_g']), row(p['ne_ln_b']),
        p['ep1_w'], row(p['ep1_b']),
        p['ep2_w'], row(p['ep2_b']), p['ep3_w'], row(p['ep3_b']),
        p['se1_w'], row(p['se1_b']),
        p['se2_w'], row(p['se2_b']),
        p['g1_w'], row(p['g1_b']), p['g2_w'], row(p['g2_b']),
        p['g3_w'], row(p['g3_b']),
        p['gr1_w'], row(p['gr1_b']), p['gr2_w'], row(p['gr2_b']),
        row(p['gr_ln_g']), row(p['gr_ln_b']),
        p['gg1_w'], row(p['gg1_b']), p['gg2_w'], row(p['gg2_b']),
        row(p['gg_ln_g']), row(p['gg_ln_b']),
        p['pn1_w'], row(p['pn1_b']), p['pn2_w'], row(p['pn2_b']),
        p['sh1_w'][:64], p['sh1_w'][64:], row(p['sh1_b']),
        p['sh2_w'], row(p['sh2_b']), row(p['sh_ln_g']), row(p['sh_ln_b']),
    ]

    node_specs = [pl.BlockSpec(memory_space=pl.ANY)]
    rem = jnp.asarray(step_count) % jnp.asarray(async_interval)
    act = (rem == 0).astype(jnp.int32).reshape(1, 1)

    param_specs = [
        pl.BlockSpec(a.shape, lambda i: (0, 0)) for a in param_args
    ] + [pl.BlockSpec((1, 1), lambda i: (0, 0))]
    out_shape = [
        jax.ShapeDtypeStruct((B, 64), jnp.float32),
        jax.ShapeDtypeStruct((B, 32), jnp.float32),
        jax.ShapeDtypeStruct((B, 1), jnp.float32),
        jax.ShapeDtypeStruct((B, _N, _N), jnp.float32),
        jax.ShapeDtypeStruct((B, _N, _N), jnp.float32),
    ]
    out_specs = [
        pl.BlockSpec((B, 64), lambda i: (0, 0)),
        pl.BlockSpec((B, 32), lambda i: (0, 0)),
        pl.BlockSpec((B, 1), lambda i: (0, 0)),
        pl.BlockSpec((B, _N, _N), lambda i: (0, 0, 0)),
        pl.BlockSpec((B, _N, _N), lambda i: (0, 0, 0)),
    ]

    outs = pl.pallas_call(
        _fused,
        grid=(1,),
        in_specs=node_specs + param_specs,
        out_specs=out_specs,
        out_shape=out_shape,
        scratch_shapes=[
            pltpu.VMEM((B, _N, D), jnp.float32),
            pltpu.SemaphoreType.DMA,
        ],
    )(*([state_buffer] + param_args + [act]))
    return tuple(outs)


# speedup vs baseline: 6.2025x; 1.0349x over previous
import jax
import jax.numpy as jnp
from jax.experimental import pallas as pl
from jax.experimental.pallas import tpu as pltpu

_N = 16
_B = 32
_IDX = (0, 136, 272, 409, 545, 682, 818, 955, 1091, 1228, 1364, 1501,
        1637, 1774, 1910, 2047)


def _dot(a, b):
    return jax.lax.dot_general(
        a, b, (((a.ndim - 1,), (0,)), ((), ())),
        preferred_element_type=jnp.float32)


def _lnorm(x, g, b):
    mu = jnp.mean(x, axis=-1, keepdims=True)
    xc = x - mu
    v = jnp.mean(xc * xc, axis=-1, keepdims=True)
    return xc * jax.lax.rsqrt(v + 1e-5) * g + b


def _fused(*refs):
    sb_ref = refs[0]
    (ne1_w, ne1_b, ne2_w, ne2_b, ne_g, ne_bb,
     ep1_w, ep1_b, ep2_w, ep2_b, ep3_w, ep3_b,
     se1_w, se1_b, se2_w, se2_b,
     g1_w, g1_b, g2_w, g2_b, g3_w, g3_b,
     gr1_w, gr1_b, gr2_w, gr2_b, gr_g, gr_bb,
     gg1_w, gg1_b, gg2_w, gg2_b, gg_g, gg_bb,
     pn1_w, pn1_b, pn2_w, pn2_b,
     sh1a, sh1b, sh1_b, sh2_w, sh2_b, sh_g, sh_bb) = (
        r[...] for r in refs[1:1 + 45])
    act_ref = refs[1 + 45]
    strat_ref, goals_ref, pri_ref, adj_ref, str_ref = refs[1 + 46:1 + 51]
    nodes_scr, sem = refs[1 + 51:]

    BN = _B * _N
    copies = [
        pltpu.make_async_copy(
            sb_ref.at[:, pl.ds(_IDX[k], 1), :],
            nodes_scr.at[:, pl.ds(k, 1), :],
            sem,
        )
        for k in range(_N)
    ]
    for c in copies:
        c.start()
    for c in copies:
        c.wait()
    nodes = nodes_scr[...].reshape(BN, sb_ref.shape[-1])

    h = jax.nn.relu(_dot(nodes, ne1_w) + ne1_b)
    h = _dot(h, ne2_w) + ne2_b
    nf = _lnorm(h, ne_g, ne_bb)
    dh = nf.shape[-1]

    nf3 = nf.reshape(_B, _N, dh)
    left = jnp.broadcast_to(nf3[:, :, None, :], (_B, _N, _N, dh))
    right = jnp.broadcast_to(nf3[:, None, :, :], (_B, _N, _N, dh))
    pair = jnp.concatenate([left.reshape(_B * _N * _N, dh),
                            right.reshape(_B * _N * _N, dh)], axis=1)
    e1 = jax.nn.relu(_dot(pair, ep1_w) + ep1_b)
    e2 = jax.nn.relu(_dot(e1, ep2_w) + ep2_b)
    logit_e = _dot(e2, ep3_w) + ep3_b
    s1 = jax.nn.relu(_dot(pair, se1_w) + se1_b)
    logit_s = _dot(s1, se2_w) + se2_b

    le = logit_e.reshape(BN, _N)
    ls = logit_s.reshape(BN, _N)
    sub_i = jax.lax.broadcasted_iota(jnp.int32, (BN, _N), 0) % _N
    lane_j = jax.lax.broadcasted_iota(jnp.int32, (BN, _N), 1)
    offd = jnp.where(sub_i == lane_j, 0.0, 1.0)
    eye = jnp.where(sub_i == lane_j, 1.0, 0.0)
    adj2 = jax.nn.sigmoid(le) * offd
    str2 = jnp.tanh(ls) * offd

    m = jnp.where(adj2 > 0.5, 1.0, 0.0) + eye
    deg = jnp.sum(m, axis=1, keepdims=True)
    dn = jax.lax.rsqrt(deg)
    tiled = jnp.broadcast_to(m[:, None, :], (BN, _B, _N)).reshape(BN, BN)
    rowg = jax.lax.broadcasted_iota(jnp.int32, (BN, BN), 0) // _N
    colg = jax.lax.broadcasted_iota(jnp.int32, (BN, BN), 1) // _N
    bd = jnp.where(rowg == colg, tiled, 0.0)

    x = nf
    for li, (w, b) in enumerate(((g1_w, g1_b), (g2_w, g2_b), (g3_w, g3_b))):
        agg = dn * _dot(bd, dn * x)
        x = _dot(agg, w) + b
        if li < 2:
            x = jax.nn.relu(x)

    graph = jnp.mean(x.reshape(_B, _N, x.shape[-1]), axis=1)
    g = jax.nn.relu(_dot(graph, gr1_w) + gr1_b)
    g = _dot(g, gr2_w) + gr2_b
    causal = _lnorm(g, gr_g, gr_bb)

    gg = jax.nn.relu(_dot(causal, gg1_w) + gg1_b)
    gg = _dot(gg, gg2_w) + gg2_b
    goals = _lnorm(gg, gg_g, gg_bb)

    pr = jax.nn.relu(_dot(causal, pn1_w) + pn1_b)
    pri = jax.nn.softplus(_dot(pr, pn2_w) + pn2_b)

    sh = jax.nn.relu(_dot(causal, sh1a) + _dot(goals, sh1b) + sh1_b)
    sh = _dot(sh, sh2_w) + sh2_b
    strat = _lnorm(sh, sh_g, sh_bb)

    active = act_ref[0, 0] == 1

    def gate(o):
        return jnp.where(active, o, jnp.full_like(o, jnp.nan))

    strat_ref[...] = gate(strat)
    goals_ref[...] = gate(goals)
    pri_ref[...] = gate(pri)
    adj_ref[...] = gate(adj2.reshape(_B, _N, _N))
    str_ref[...] = gate(str2.reshape(_B, _N, _N))


def kernel(state_buffer, params, step_count, async_interval):
    p = params
    B, S, D = state_buffer.shape

    def row(v):
        return v.reshape(1, -1)

    param_args = [
        p['ne1_w'], row(p['ne1_b']), p['ne2_w'], row(p['ne2_b']),
        row(p['ne_ln_g']), row(p['ne_ln_b']),
        p['ep1_w'], row(p['ep1_b']),
        p['ep2_w'], row(p['ep2_b']), p['ep3_w'], row(p['ep3_b']),
        p['se1_w'], row(p['se1_b']),
        p['se2_w'], row(p['se2_b']),
        p['g1_w'], row(p['g1_b']), p['g2_w'], row(p['g2_b']),
        p['g3_w'], row(p['g3_b']),
        p['gr1_w'], row(p['gr1_b']), p['gr2_w'], row(p['gr2_b']),
        row(p['gr_ln_g']), row(p['gr_ln_b']),
        p['gg1_w'], row(p['gg1_b']), p['gg2_w'], row(p['gg2_b']),
        row(p['gg_ln_g']), row(p['gg_ln_b']),
        p['pn1_w'], row(p['pn1_b']), p['pn2_w'], row(p['pn2_b']),
        p['sh1_w'][:64], p['sh1_w'][64:], row(p['sh1_b']),
        p['sh2_w'], row(p['sh2_b']), row(p['sh_ln_g']), row(p['sh_ln_b']),
    ]

    node_specs = [pl.BlockSpec(memory_space=pl.ANY)]
    rem = jnp.asarray(step_count) % jnp.asarray(async_interval)
    act = (rem == 0).astype(jnp.int32).reshape(1, 1)

    param_specs = [
        pl.BlockSpec(a.shape, lambda i: (0, 0)) for a in param_args
    ] + [pl.BlockSpec((1, 1), lambda i: (0, 0))]
    out_shape = [
        jax.ShapeDtypeStruct((B, 64), jnp.float32),
        jax.ShapeDtypeStruct((B, 32), jnp.float32),
        jax.ShapeDtypeStruct((B, 1), jnp.float32),
        jax.ShapeDtypeStruct((B, _N, _N), jnp.float32),
        jax.ShapeDtypeStruct((B, _N, _N), jnp.float32),
    ]
    out_specs = [
        pl.BlockSpec((B, 64), lambda i: (0, 0)),
        pl.BlockSpec((B, 32), lambda i: (0, 0)),
        pl.BlockSpec((B, 1), lambda i: (0, 0)),
        pl.BlockSpec((B, _N, _N), lambda i: (0, 0, 0)),
        pl.BlockSpec((B, _N, _N), lambda i: (0, 0, 0)),
    ]

    outs = pl.pallas_call(
        _fused,
        grid=(1,),
        in_specs=node_specs + param_specs,
        out_specs=out_specs,
        out_shape=out_shape,
        scratch_shapes=[
            pltpu.VMEM((B, _N, D), jnp.float32),
            pltpu.SemaphoreType.DMA,
        ],
    )(*([state_buffer] + param_args + [act]))
    return tuple(outs)
